# baseline (speedup 1.0000x reference)
import functools

import numpy as np
import jax
import jax.numpy as jnp
from jax.experimental import pallas as pl
from jax.experimental.pallas import tpu as pltpu

_LANES = 128


def _fused_kernel(src_ref, x_ref, y_ref, z_ref, sc_ref, sem_ref, vm_ref,
                  cf_ref, feat_ref, fout_ref, corner_ref, param_ref,
                  scr_ref, msk_ref, sems_ref,
                  *, r_chunk, n_src, score_thre):
    xmn = jnp.min(x_ref[...], axis=1, keepdims=True)
    xmx = jnp.max(x_ref[...], axis=1, keepdims=True)
    ymn = jnp.min(y_ref[...], axis=1, keepdims=True)
    ymx = jnp.max(y_ref[...], axis=1, keepdims=True)
    zmn = jnp.min(z_ref[...], axis=1, keepdims=True)
    zmx = jnp.max(z_ref[...], axis=1, keepdims=True)

    cx = (xmn + xmx) * 0.5
    cy = (ymn + ymx) * 0.5
    cz = (zmn + zmx) * 0.5
    dx = xmx - xmn
    dy = ymx - ymn
    dz = zmx - zmn

    sig = jax.nn.sigmoid(sc_ref[...])
    msk = (sig > score_thre).astype(jnp.float32)
    sem = sem_ref[...]

    cm = cf_ref[...]
    packed = (cx * cm[0:1, :] + cy * cm[1:2, :] + cz * cm[2:3, :]
              + dx * cm[3:4, :] + dy * cm[4:5, :] + dz * cm[5:6, :]
              + sig * cm[6:7, :] + msk * cm[7:8, :] + sem * cm[8:9, :])
    pm = packed * vm_ref[...]
    corner_ref[...] = pm[:, 0:24]
    param_ref[...] = pm[:, 24:30]
    scr_ref[...] = pm[:, 30:31]
    msk_ref[...] = pm[:, 31:32]
    sems_ref[...] = pm[:, 32:33]

    base = pl.program_id(1) * r_chunk
    for i in range(r_chunk):
        idx = src_ref[base + i]
        safe = jnp.minimum(idx, n_src - 1)
        vf = (idx < n_src).astype(jnp.float32)
        b8 = pl.multiple_of((safe >> 3) << 3, 8)
        chunk = feat_ref[pl.ds(b8, 8), :]
        row = pltpu.roll(chunk, -(safe & 7), axis=0)[0:1, :]
        fout_ref[pl.ds(i, 1), :] = row * vf


def _build_coeff():
    xs = np.array([1, 1, -1, -1, 1, 1, -1, -1], np.float32)
    ys = np.array([1, 1, 1, 1, -1, -1, -1, -1], np.float32)
    zs = np.array([1, -1, -1, 1, 1, -1, -1, 1], np.float32)
    c = np.zeros((16, _LANES), np.float32)
    for j in range(8):
        c[0, 3 * j + 0] = 1.0
        c[3, 3 * j + 0] = xs[j] * 0.5
        c[1, 3 * j + 1] = 1.0
        c[5, 3 * j + 1] = ys[j] * 0.5
        c[2, 3 * j + 2] = 1.0
        c[4, 3 * j + 2] = zs[j] * 0.5
    for j in range(6):
        c[j, 24 + j] = 1.0
    c[6, 30] = 1.0
    c[7, 31] = 1.0
    c[8, 32] = 1.0
    return jnp.asarray(c)


def _run_fused(src_rows, xs, ys, zs, scores, sems, vmask, coeff, feats,
               r_chunk, score_thre):
    n_src, c = feats.shape
    rpad, k = xs.shape
    ch = c // 2
    coord_spec = pl.BlockSpec((r_chunk, k), lambda h, r, src: (r, 0))
    col_spec = pl.BlockSpec((r_chunk, 1), lambda h, r, src: (r, 0))
    grid_spec = pltpu.PrefetchScalarGridSpec(
        num_scalar_prefetch=1,
        grid=(2, rpad // r_chunk),
        in_specs=[
            coord_spec, coord_spec, coord_spec, col_spec, col_spec, col_spec,
            pl.BlockSpec((16, _LANES), lambda h, r, src: (0, 0)),
            pl.BlockSpec((n_src, ch), lambda h, r, src: (0, h)),
        ],
        out_specs=[
            pl.BlockSpec((r_chunk, ch), lambda h, r, src: (r, h)),
            pl.BlockSpec((r_chunk, 24), lambda h, r, src: (r, 0)),
            pl.BlockSpec((r_chunk, 6), lambda h, r, src: (r, 0)),
            pl.BlockSpec((r_chunk, 1), lambda h, r, src: (r, 0)),
            pl.BlockSpec((r_chunk, 1), lambda h, r, src: (r, 0)),
            pl.BlockSpec((r_chunk, 1), lambda h, r, src: (r, 0)),
        ],
    )
    return pl.pallas_call(
        functools.partial(_fused_kernel, r_chunk=r_chunk, n_src=n_src,
                          score_thre=score_thre),
        out_shape=[jax.ShapeDtypeStruct((rpad, c), jnp.float32),
                   jax.ShapeDtypeStruct((rpad, 24), jnp.float32),
                   jax.ShapeDtypeStruct((rpad, 6), jnp.float32),
                   jax.ShapeDtypeStruct((rpad, 1), jnp.float32),
                   jax.ShapeDtypeStruct((rpad, 1), jnp.float32),
                   jax.ShapeDtypeStruct((rpad, 1), jnp.float32)],
        grid_spec=grid_spec,
        compiler_params=pltpu.CompilerParams(
            dimension_semantics=("parallel", "arbitrary"),
            vmem_limit_bytes=48 * 1024 * 1024),
    )(src_rows, xs, ys, zs, scores, sems, vmask, coeff, feats)


def _capnet(locs_float, proposal_feats, proposals_idx, proposals_offset,
            proposal_scores, semantic_preds, batch_offsets,
            batch_size, num_proposal, score_thre):
    P = int(proposals_offset.shape[0]) - 1
    M = int(proposals_idx.shape[0])
    C = int(proposal_feats.shape[1])
    K = M // P

    first_pts = proposals_idx.reshape(P, K, 2)[:, 0, 1]
    N = int(locs_float.shape[0])
    batch_id = (first_pts // (N // batch_size)).astype(jnp.int32)
    onehot = (batch_id[:, None] ==
              jnp.arange(batch_size, dtype=jnp.int32)[None, :]).astype(jnp.int32)
    cum = jnp.cumsum(onehot, axis=0)
    slot = jnp.sum(cum * onehot, axis=1) - 1
    valid_slot = slot < num_proposal

    R = batch_size * num_proposal
    rows = batch_id * num_proposal + slot
    scatter_rows = jnp.where(valid_slot, rows, R)
    src_rows = jnp.full((R,), P, jnp.int32).at[scatter_rows].set(
        jnp.arange(P, dtype=jnp.int32), mode="drop")

    src_safe = jnp.minimum(src_rows, P - 1)
    idx2 = (src_safe[:, None] * K +
            jnp.arange(K, dtype=jnp.int32)[None, :])
    mem_ids = proposals_idx[idx2, 1]

    xs = locs_float[mem_ids, 0]
    ys = locs_float[mem_ids, 1]
    zs = locs_float[mem_ids, 2]

    small_tab = jnp.concatenate([
        proposal_scores.reshape(P).astype(jnp.float32),
        semantic_preds.astype(jnp.float32),
    ])
    sidx = jnp.concatenate([src_safe, mem_ids[:, 0] + P])
    sg = small_tab[sidx]
    scores = sg[:R].reshape(R, 1)
    sems = sg[R:].reshape(R, 1)
    vmask = (src_rows < P).astype(jnp.float32).reshape(R, 1)

    r_chunk = 64
    while R % r_chunk:
        r_chunk //= 2

    fout, corner, params, scr, msk, sem_o = _run_fused(
        src_rows, xs, ys, zs, scores, sems, vmask,
        _build_coeff(), proposal_feats.astype(jnp.float32), r_chunk,
        score_thre)

    out = {}
    out["bbox_feature"] = fout.reshape(batch_size, num_proposal, C)
    out["bbox_corner"] = corner.reshape(batch_size, num_proposal, 8, 3)
    out["bbox_parameters"] = params.reshape(batch_size, num_proposal, 6)
    out["bbox_scores"] = scr.reshape(batch_size, num_proposal)
    out["bbox_mask"] = msk.reshape(batch_size, num_proposal)
    out["bbox_sems"] = sem_o.reshape(batch_size, num_proposal)
    out["sem_cls"] = out["bbox_sems"]
    return out


def kernel(locs_float, proposal_feats, proposals_idx, proposals_offset,
           proposal_scores, semantic_preds, batch_offsets):
    return _capnet(locs_float, proposal_feats, proposals_idx, proposals_offset,
                   proposal_scores, semantic_preds, batch_offsets,
                   batch_size=8, num_proposal=256, score_thre=0.09)


# speedup vs baseline: 1.2145x; 1.2145x over previous
import functools

import numpy as np
import jax
import jax.numpy as jnp
from jax.experimental import pallas as pl
from jax.experimental.pallas import tpu as pltpu

_LANES = 128


def _fused_kernel(src_ref, x_ref, y_ref, z_ref, sc_ref, sem_ref, vm_ref,
                  cf_ref, feat_ref, fout_ref, corner_ref, param_ref,
                  scr_ref, msk_ref, sems_ref,
                  *, r_chunk, n_src, score_thre):
    xmn = jnp.min(x_ref[...], axis=1, keepdims=True)
    xmx = jnp.max(x_ref[...], axis=1, keepdims=True)
    ymn = jnp.min(y_ref[...], axis=1, keepdims=True)
    ymx = jnp.max(y_ref[...], axis=1, keepdims=True)
    zmn = jnp.min(z_ref[...], axis=1, keepdims=True)
    zmx = jnp.max(z_ref[...], axis=1, keepdims=True)

    cx = (xmn + xmx) * 0.5
    cy = (ymn + ymx) * 0.5
    cz = (zmn + zmx) * 0.5
    dx = xmx - xmn
    dy = ymx - ymn
    dz = zmx - zmn

    sig = jax.nn.sigmoid(sc_ref[...])
    msk = (sig > score_thre).astype(jnp.float32)
    sem = sem_ref[...]

    cm = cf_ref[...]
    packed = (cx * cm[0:1, :] + cy * cm[1:2, :] + cz * cm[2:3, :]
              + dx * cm[3:4, :] + dy * cm[4:5, :] + dz * cm[5:6, :]
              + sig * cm[6:7, :] + msk * cm[7:8, :] + sem * cm[8:9, :])
    pm = packed * vm_ref[...]
    corner_ref[...] = pm[:, 0:24]
    param_ref[...] = pm[:, 24:30]
    scr_ref[...] = pm[:, 30:31]
    msk_ref[...] = pm[:, 31:32]
    sems_ref[...] = pm[:, 32:33]

    base = pl.program_id(0) * r_chunk
    for i in range(r_chunk):
        idx = src_ref[base + i]
        safe = jnp.minimum(idx, n_src - 1)
        vf = (idx < n_src).astype(jnp.float32)
        b8 = pl.multiple_of((safe >> 3) << 3, 8)
        chunk = feat_ref[pl.ds(b8, 8), :]
        row = pltpu.roll(chunk, -(safe & 7), axis=0)[0:1, :]
        fout_ref[pl.ds(i, 1), :] = row * vf


def _build_coeff():
    xs = np.array([1, 1, -1, -1, 1, 1, -1, -1], np.float32)
    ys = np.array([1, 1, 1, 1, -1, -1, -1, -1], np.float32)
    zs = np.array([1, -1, -1, 1, 1, -1, -1, 1], np.float32)
    c = np.zeros((16, _LANES), np.float32)
    for j in range(8):
        c[0, 3 * j + 0] = 1.0
        c[3, 3 * j + 0] = xs[j] * 0.5
        c[1, 3 * j + 1] = 1.0
        c[5, 3 * j + 1] = ys[j] * 0.5
        c[2, 3 * j + 2] = 1.0
        c[4, 3 * j + 2] = zs[j] * 0.5
    for j in range(6):
        c[j, 24 + j] = 1.0
    c[6, 30] = 1.0
    c[7, 31] = 1.0
    c[8, 32] = 1.0
    return jnp.asarray(c)


def _run_fused(src_rows, xs, ys, zs, scores, sems, vmask, coeff, feats,
               r_chunk, score_thre):
    n_src, c = feats.shape
    rpad, k = xs.shape
    coord_spec = pl.BlockSpec((r_chunk, k), lambda r, src: (r, 0))
    col_spec = pl.BlockSpec((r_chunk, 1), lambda r, src: (r, 0))
    grid_spec = pltpu.PrefetchScalarGridSpec(
        num_scalar_prefetch=1,
        grid=(rpad // r_chunk,),
        in_specs=[
            coord_spec, coord_spec, coord_spec, col_spec, col_spec, col_spec,
            pl.BlockSpec((16, _LANES), lambda r, src: (0, 0)),
            pl.BlockSpec((n_src, c), lambda r, src: (0, 0)),
        ],
        out_specs=[
            pl.BlockSpec((r_chunk, c), lambda r, src: (r, 0)),
            pl.BlockSpec((r_chunk, 24), lambda r, src: (r, 0)),
            pl.BlockSpec((r_chunk, 6), lambda r, src: (r, 0)),
            pl.BlockSpec((r_chunk, 1), lambda r, src: (r, 0)),
            pl.BlockSpec((r_chunk, 1), lambda r, src: (r, 0)),
            pl.BlockSpec((r_chunk, 1), lambda r, src: (r, 0)),
        ],
    )
    return pl.pallas_call(
        functools.partial(_fused_kernel, r_chunk=r_chunk, n_src=n_src,
                          score_thre=score_thre),
        out_shape=[jax.ShapeDtypeStruct((rpad, c), jnp.float32),
                   jax.ShapeDtypeStruct((rpad, 24), jnp.float32),
                   jax.ShapeDtypeStruct((rpad, 6), jnp.float32),
                   jax.ShapeDtypeStruct((rpad, 1), jnp.float32),
                   jax.ShapeDtypeStruct((rpad, 1), jnp.float32),
                   jax.ShapeDtypeStruct((rpad, 1), jnp.float32)],
        grid_spec=grid_spec,
        compiler_params=pltpu.CompilerParams(
            dimension_semantics=("parallel",),
            vmem_limit_bytes=48 * 1024 * 1024),
    )(src_rows, xs, ys, zs, scores, sems, vmask, coeff, feats)


def _capnet(locs_float, proposal_feats, proposals_idx, proposals_offset,
            proposal_scores, semantic_preds, batch_offsets,
            batch_size, num_proposal, score_thre):
    P = int(proposals_offset.shape[0]) - 1
    M = int(proposals_idx.shape[0])
    C = int(proposal_feats.shape[1])
    K = M // P

    first_pts = proposals_idx.reshape(P, K, 2)[:, 0, 1]
    N = int(locs_float.shape[0])
    batch_id = (first_pts // (N // batch_size)).astype(jnp.int32)
    onehot = (batch_id[:, None] ==
              jnp.arange(batch_size, dtype=jnp.int32)[None, :]).astype(jnp.int32)
    cum = jnp.cumsum(onehot, axis=0)
    slot = jnp.sum(cum * onehot, axis=1) - 1
    valid_slot = slot < num_proposal

    R = batch_size * num_proposal
    rows = batch_id * num_proposal + slot
    scatter_rows = jnp.where(valid_slot, rows, R)
    src_rows = jnp.full((R,), P, jnp.int32).at[scatter_rows].set(
        jnp.arange(P, dtype=jnp.int32), mode="drop")

    src_safe = jnp.minimum(src_rows, P - 1)
    idx2 = (src_safe[:, None] * K +
            jnp.arange(K, dtype=jnp.int32)[None, :])
    mem_ids = proposals_idx[idx2, 1]

    xs = locs_float[mem_ids, 0]
    ys = locs_float[mem_ids, 1]
    zs = locs_float[mem_ids, 2]

    small_tab = jnp.concatenate([
        proposal_scores.reshape(P).astype(jnp.float32),
        semantic_preds.astype(jnp.float32),
    ])
    sidx = jnp.concatenate([src_safe, mem_ids[:, 0] + P])
    sg = small_tab[sidx]
    scores = sg[:R].reshape(R, 1)
    sems = sg[R:].reshape(R, 1)
    vmask = (src_rows < P).astype(jnp.float32).reshape(R, 1)

    r_chunk = 128
    while R % r_chunk:
        r_chunk //= 2

    fout, corner, params, scr, msk, sem_o = _run_fused(
        src_rows, xs, ys, zs, scores, sems, vmask,
        _build_coeff(), proposal_feats.astype(jnp.float32), r_chunk,
        score_thre)

    out = {}
    out["bbox_feature"] = fout.reshape(batch_size, num_proposal, C)
    out["bbox_corner"] = corner.reshape(batch_size, num_proposal, 8, 3)
    out["bbox_parameters"] = params.reshape(batch_size, num_proposal, 6)
    out["bbox_scores"] = scr.reshape(batch_size, num_proposal)
    out["bbox_mask"] = msk.reshape(batch_size, num_proposal)
    out["bbox_sems"] = sem_o.reshape(batch_size, num_proposal)
    out["sem_cls"] = out["bbox_sems"]
    return out


def kernel(locs_float, proposal_feats, proposals_idx, proposals_offset,
           proposal_scores, semantic_preds, batch_offsets):
    return _capnet(locs_float, proposal_feats, proposals_idx, proposals_offset,
                   proposal_scores, semantic_preds, batch_offsets,
                   batch_size=8, num_proposal=256, score_thre=0.09)


# speedup vs baseline: 1.2341x; 1.0162x over previous
import functools

import numpy as np
import jax
import jax.numpy as jnp
from jax.experimental import pallas as pl
from jax.experimental.pallas import tpu as pltpu

_LANES = 128


def _fused_kernel(src_ref, x_ref, y_ref, z_ref, sc_ref, sem_ref, vm_ref,
                  cf_ref, feat_ref, fout_ref, corner_ref, param_ref,
                  scr_ref, msk_ref, sems_ref,
                  *, r_chunk, n_src, score_thre):
    xmn = jnp.min(x_ref[...], axis=1, keepdims=True)
    xmx = jnp.max(x_ref[...], axis=1, keepdims=True)
    ymn = jnp.min(y_ref[...], axis=1, keepdims=True)
    ymx = jnp.max(y_ref[...], axis=1, keepdims=True)
    zmn = jnp.min(z_ref[...], axis=1, keepdims=True)
    zmx = jnp.max(z_ref[...], axis=1, keepdims=True)

    cx = (xmn + xmx) * 0.5
    cy = (ymn + ymx) * 0.5
    cz = (zmn + zmx) * 0.5
    dx = xmx - xmn
    dy = ymx - ymn
    dz = zmx - zmn

    sig = jax.nn.sigmoid(sc_ref[...])
    msk = (sig > score_thre).astype(jnp.float32)
    sem = sem_ref[...]

    cm = cf_ref[...]
    packed = (cx * cm[0:1, :] + cy * cm[1:2, :] + cz * cm[2:3, :]
              + dx * cm[3:4, :] + dy * cm[4:5, :] + dz * cm[5:6, :]
              + sig * cm[6:7, :] + msk * cm[7:8, :] + sem * cm[8:9, :])
    pm = packed * vm_ref[...]
    corner_ref[...] = pm[:, 0:24]
    param_ref[...] = pm[:, 24:30]
    scr_ref[...] = pm[:, 30:31]
    msk_ref[...] = pm[:, 31:32]
    sems_ref[...] = pm[:, 32:33]

    base = pl.program_id(0) * r_chunk
    for i in range(r_chunk):
        idx = src_ref[base + i]
        safe = jnp.minimum(idx, n_src - 1)
        vf = (idx < n_src).astype(jnp.float32)
        b8 = pl.multiple_of((safe >> 3) << 3, 8)
        chunk = feat_ref[pl.ds(b8, 8), :]
        row = pltpu.roll(chunk, -(safe & 7), axis=0)[0:1, :]
        fout_ref[pl.ds(i, 1), :] = row * vf


def _build_coeff():
    xs = np.array([1, 1, -1, -1, 1, 1, -1, -1], np.float32)
    ys = np.array([1, 1, 1, 1, -1, -1, -1, -1], np.float32)
    zs = np.array([1, -1, -1, 1, 1, -1, -1, 1], np.float32)
    c = np.zeros((16, _LANES), np.float32)
    for j in range(8):
        c[0, 3 * j + 0] = 1.0
        c[3, 3 * j + 0] = xs[j] * 0.5
        c[1, 3 * j + 1] = 1.0
        c[5, 3 * j + 1] = ys[j] * 0.5
        c[2, 3 * j + 2] = 1.0
        c[4, 3 * j + 2] = zs[j] * 0.5
    for j in range(6):
        c[j, 24 + j] = 1.0
    c[6, 30] = 1.0
    c[7, 31] = 1.0
    c[8, 32] = 1.0
    return jnp.asarray(c)


def _run_fused(src_rows, xs, ys, zs, scores, sems, vmask, coeff, feats,
               r_chunk, score_thre):
    n_src, c = feats.shape
    rpad, k = xs.shape
    coord_spec = pl.BlockSpec((r_chunk, k), lambda r, src: (r, 0))
    col_spec = pl.BlockSpec((r_chunk, 1), lambda r, src: (r, 0))
    grid_spec = pltpu.PrefetchScalarGridSpec(
        num_scalar_prefetch=1,
        grid=(rpad // r_chunk,),
        in_specs=[
            coord_spec, coord_spec, coord_spec, col_spec, col_spec, col_spec,
            pl.BlockSpec((16, _LANES), lambda r, src: (0, 0)),
            pl.BlockSpec((n_src, c), lambda r, src: (0, 0)),
        ],
        out_specs=[
            pl.BlockSpec((r_chunk, c), lambda r, src: (r, 0)),
            pl.BlockSpec((r_chunk, 24), lambda r, src: (r, 0)),
            pl.BlockSpec((r_chunk, 6), lambda r, src: (r, 0)),
            pl.BlockSpec((r_chunk, 1), lambda r, src: (r, 0)),
            pl.BlockSpec((r_chunk, 1), lambda r, src: (r, 0)),
            pl.BlockSpec((r_chunk, 1), lambda r, src: (r, 0)),
        ],
    )
    return pl.pallas_call(
        functools.partial(_fused_kernel, r_chunk=r_chunk, n_src=n_src,
                          score_thre=score_thre),
        out_shape=[jax.ShapeDtypeStruct((rpad, c), jnp.float32),
                   jax.ShapeDtypeStruct((rpad, 24), jnp.float32),
                   jax.ShapeDtypeStruct((rpad, 6), jnp.float32),
                   jax.ShapeDtypeStruct((rpad, 1), jnp.float32),
                   jax.ShapeDtypeStruct((rpad, 1), jnp.float32),
                   jax.ShapeDtypeStruct((rpad, 1), jnp.float32)],
        grid_spec=grid_spec,
        compiler_params=pltpu.CompilerParams(
            dimension_semantics=("parallel",),
            vmem_limit_bytes=48 * 1024 * 1024),
    )(src_rows, xs, ys, zs, scores, sems, vmask, coeff, feats)


def _capnet(locs_float, proposal_feats, proposals_idx, proposals_offset,
            proposal_scores, semantic_preds, batch_offsets,
            batch_size, num_proposal, score_thre):
    P = int(proposals_offset.shape[0]) - 1
    M = int(proposals_idx.shape[0])
    C = int(proposal_feats.shape[1])
    K = M // P

    first_pts = proposals_idx.reshape(P, K, 2)[:, 0, 1]
    N = int(locs_float.shape[0])
    batch_id = (first_pts // (N // batch_size)).astype(jnp.int32)
    onehot = (batch_id[:, None] ==
              jnp.arange(batch_size, dtype=jnp.int32)[None, :]).astype(jnp.int32)
    cum = jnp.cumsum(onehot, axis=0)
    slot = jnp.sum(cum * onehot, axis=1) - 1
    valid_slot = slot < num_proposal

    R = batch_size * num_proposal
    rows = batch_id * num_proposal + slot
    scatter_rows = jnp.where(valid_slot, rows, R)
    src_rows = jnp.full((R,), P, jnp.int32).at[scatter_rows].set(
        jnp.arange(P, dtype=jnp.int32), mode="drop")

    src_safe = jnp.minimum(src_rows, P - 1)
    idx2 = (src_safe[:, None] * K +
            jnp.arange(K, dtype=jnp.int32)[None, :])
    mem_ids = proposals_idx[idx2, 1]

    xs = locs_float[mem_ids, 0]
    ys = locs_float[mem_ids, 1]
    zs = locs_float[mem_ids, 2]

    small_tab = jnp.concatenate([
        proposal_scores.reshape(P).astype(jnp.float32),
        semantic_preds.astype(jnp.float32),
    ])
    sidx = jnp.concatenate([src_safe, mem_ids[:, 0] + P])
    sg = small_tab[sidx]
    scores = sg[:R].reshape(R, 1)
    sems = sg[R:].reshape(R, 1)
    vmask = (src_rows < P).astype(jnp.float32).reshape(R, 1)

    r_chunk = 256
    while R % r_chunk:
        r_chunk //= 2

    fout, corner, params, scr, msk, sem_o = _run_fused(
        src_rows, xs, ys, zs, scores, sems, vmask,
        _build_coeff(), proposal_feats.astype(jnp.float32), r_chunk,
        score_thre)

    out = {}
    out["bbox_feature"] = fout.reshape(batch_size, num_proposal, C)
    out["bbox_corner"] = corner.reshape(batch_size, num_proposal, 8, 3)
    out["bbox_parameters"] = params.reshape(batch_size, num_proposal, 6)
    out["bbox_scores"] = scr.reshape(batch_size, num_proposal)
    out["bbox_mask"] = msk.reshape(batch_size, num_proposal)
    out["bbox_sems"] = sem_o.reshape(batch_size, num_proposal)
    out["sem_cls"] = out["bbox_sems"]
    return out


def kernel(locs_float, proposal_feats, proposals_idx, proposals_offset,
           proposal_scores, semantic_preds, batch_offsets):
    return _capnet(locs_float, proposal_feats, proposals_idx, proposals_offset,
                   proposal_scores, semantic_preds, batch_offsets,
                   batch_size=8, num_proposal=256, score_thre=0.09)


# speedup vs baseline: 1.2344x; 1.0003x over previous
import functools

import numpy as np
import jax
import jax.numpy as jnp
from jax.experimental import pallas as pl
from jax.experimental.pallas import tpu as pltpu

_LANES = 128


def _fused_kernel(src_ref, x_ref, y_ref, z_ref, sc_ref, sem_ref, vm_ref,
                  cf_ref, feat_ref, fout_ref, corner_ref, param_ref,
                  scr_ref, msk_ref, sems_ref,
                  *, r_chunk, n_src, score_thre):
    xmn = jnp.min(x_ref[...], axis=1, keepdims=True)
    xmx = jnp.max(x_ref[...], axis=1, keepdims=True)
    ymn = jnp.min(y_ref[...], axis=1, keepdims=True)
    ymx = jnp.max(y_ref[...], axis=1, keepdims=True)
    zmn = jnp.min(z_ref[...], axis=1, keepdims=True)
    zmx = jnp.max(z_ref[...], axis=1, keepdims=True)

    cx = (xmn + xmx) * 0.5
    cy = (ymn + ymx) * 0.5
    cz = (zmn + zmx) * 0.5
    dx = xmx - xmn
    dy = ymx - ymn
    dz = zmx - zmn

    sig = jax.nn.sigmoid(sc_ref[...])
    msk = (sig > score_thre).astype(jnp.float32)
    sem = sem_ref[...]

    cm = cf_ref[...]
    packed = (cx * cm[0:1, :] + cy * cm[1:2, :] + cz * cm[2:3, :]
              + dx * cm[3:4, :] + dy * cm[4:5, :] + dz * cm[5:6, :]
              + sig * cm[6:7, :] + msk * cm[7:8, :] + sem * cm[8:9, :])
    pm = packed * vm_ref[...]
    corner_ref[...] = pm[:, 0:24]
    param_ref[...] = pm[:, 24:30]
    scr_ref[...] = pm[:, 30:31]
    msk_ref[...] = pm[:, 31:32]
    sems_ref[...] = pm[:, 32:33]

    base = pl.program_id(0) * r_chunk
    for i in range(r_chunk):
        idx = src_ref[base + i]
        safe = jnp.minimum(idx, n_src - 1)
        vf = (idx < n_src).astype(jnp.float32)
        b8 = pl.multiple_of((safe >> 3) << 3, 8)
        chunk = feat_ref[pl.ds(b8, 8), :]
        row = pltpu.roll(chunk, -(safe & 7), axis=0)[0:1, :]
        fout_ref[pl.ds(i, 1), :] = row * vf


def _build_coeff():
    xs = np.array([1, 1, -1, -1, 1, 1, -1, -1], np.float32)
    ys = np.array([1, 1, 1, 1, -1, -1, -1, -1], np.float32)
    zs = np.array([1, -1, -1, 1, 1, -1, -1, 1], np.float32)
    c = np.zeros((16, _LANES), np.float32)
    for j in range(8):
        c[0, 3 * j + 0] = 1.0
        c[3, 3 * j + 0] = xs[j] * 0.5
        c[1, 3 * j + 1] = 1.0
        c[5, 3 * j + 1] = ys[j] * 0.5
        c[2, 3 * j + 2] = 1.0
        c[4, 3 * j + 2] = zs[j] * 0.5
    for j in range(6):
        c[j, 24 + j] = 1.0
    c[6, 30] = 1.0
    c[7, 31] = 1.0
    c[8, 32] = 1.0
    return jnp.asarray(c)


def _run_fused(src_rows, xs, ys, zs, scores, sems, vmask, coeff, feats,
               r_chunk, score_thre):
    n_src, c = feats.shape
    rpad, k = xs.shape
    coord_spec = pl.BlockSpec((r_chunk, k), lambda r, src: (r, 0))
    col_spec = pl.BlockSpec((r_chunk, 1), lambda r, src: (r, 0))
    grid_spec = pltpu.PrefetchScalarGridSpec(
        num_scalar_prefetch=1,
        grid=(rpad // r_chunk,),
        in_specs=[
            coord_spec, coord_spec, coord_spec, col_spec, col_spec, col_spec,
            pl.BlockSpec((16, _LANES), lambda r, src: (0, 0)),
            pl.BlockSpec((n_src, c), lambda r, src: (0, 0)),
        ],
        out_specs=[
            pl.BlockSpec((r_chunk, c), lambda r, src: (r, 0)),
            pl.BlockSpec((r_chunk, 24), lambda r, src: (r, 0)),
            pl.BlockSpec((r_chunk, 6), lambda r, src: (r, 0)),
            pl.BlockSpec((r_chunk, 1), lambda r, src: (r, 0)),
            pl.BlockSpec((r_chunk, 1), lambda r, src: (r, 0)),
            pl.BlockSpec((r_chunk, 1), lambda r, src: (r, 0)),
        ],
    )
    return pl.pallas_call(
        functools.partial(_fused_kernel, r_chunk=r_chunk, n_src=n_src,
                          score_thre=score_thre),
        out_shape=[jax.ShapeDtypeStruct((rpad, c), jnp.float32),
                   jax.ShapeDtypeStruct((rpad, 24), jnp.float32),
                   jax.ShapeDtypeStruct((rpad, 6), jnp.float32),
                   jax.ShapeDtypeStruct((rpad, 1), jnp.float32),
                   jax.ShapeDtypeStruct((rpad, 1), jnp.float32),
                   jax.ShapeDtypeStruct((rpad, 1), jnp.float32)],
        grid_spec=grid_spec,
        compiler_params=pltpu.CompilerParams(
            dimension_semantics=("parallel",),
            vmem_limit_bytes=48 * 1024 * 1024),
    )(src_rows, xs, ys, zs, scores, sems, vmask, coeff, feats)


def _capnet(locs_float, proposal_feats, proposals_idx, proposals_offset,
            proposal_scores, semantic_preds, batch_offsets,
            batch_size, num_proposal, score_thre):
    P = int(proposals_offset.shape[0]) - 1
    M = int(proposals_idx.shape[0])
    C = int(proposal_feats.shape[1])
    K = M // P

    first_pts = proposals_idx.reshape(P, K, 2)[:, 0, 1]
    N = int(locs_float.shape[0])
    batch_id = (first_pts // (N // batch_size)).astype(jnp.int32)
    onehot = (batch_id[:, None] ==
              jnp.arange(batch_size, dtype=jnp.int32)[None, :]).astype(jnp.int32)
    cum = jnp.cumsum(onehot, axis=0)
    slot = jnp.sum(cum * onehot, axis=1) - 1
    valid_slot = slot < num_proposal

    R = batch_size * num_proposal
    rows = batch_id * num_proposal + slot
    scatter_rows = jnp.where(valid_slot, rows, R)
    src_rows = jnp.full((R,), P, jnp.int32).at[scatter_rows].set(
        jnp.arange(P, dtype=jnp.int32), mode="drop")

    src_safe = jnp.minimum(src_rows, P - 1)
    idx2 = (src_safe[:, None] * K +
            jnp.arange(K, dtype=jnp.int32)[None, :])
    mem_ids = proposals_idx[idx2, 1]

    xs = locs_float[mem_ids, 0]
    ys = locs_float[mem_ids, 1]
    zs = locs_float[mem_ids, 2]

    small_tab = jnp.concatenate([
        proposal_scores.reshape(P).astype(jnp.float32),
        semantic_preds.astype(jnp.float32),
    ])
    sidx = jnp.concatenate([src_safe, mem_ids[:, 0] + P])
    sg = small_tab[sidx]
    scores = sg[:R].reshape(R, 1)
    sems = sg[R:].reshape(R, 1)
    vmask = (src_rows < P).astype(jnp.float32).reshape(R, 1)

    r_chunk = 512
    while R % r_chunk:
        r_chunk //= 2

    fout, corner, params, scr, msk, sem_o = _run_fused(
        src_rows, xs, ys, zs, scores, sems, vmask,
        _build_coeff(), proposal_feats.astype(jnp.float32), r_chunk,
        score_thre)

    out = {}
    out["bbox_feature"] = fout.reshape(batch_size, num_proposal, C)
    out["bbox_corner"] = corner.reshape(batch_size, num_proposal, 8, 3)
    out["bbox_parameters"] = params.reshape(batch_size, num_proposal, 6)
    out["bbox_scores"] = scr.reshape(batch_size, num_proposal)
    out["bbox_mask"] = msk.reshape(batch_size, num_proposal)
    out["bbox_sems"] = sem_o.reshape(batch_size, num_proposal)
    out["sem_cls"] = out["bbox_sems"]
    return out


def kernel(locs_float, proposal_feats, proposals_idx, proposals_offset,
           proposal_scores, semantic_preds, batch_offsets):
    return _capnet(locs_float, proposal_feats, proposals_idx, proposals_offset,
                   proposal_scores, semantic_preds, batch_offsets,
                   batch_size=8, num_proposal=256, score_thre=0.09)
